# baseline (speedup 1.0000x reference)
import functools

import jax
import jax.numpy as jnp
from jax import lax
from jax.experimental import pallas as pl
from jax.experimental.pallas import tpu as pltpu
from jax.experimental.pallas import tpu_sc as plsc

N = 10000
F = 128
H = 64
E = 320000
NC = 2
NS = 16
NW = NC * NS
K = 128
NBLK = E // K
BPW = NBLK // NW
NP = 10240
RPS = NP // NS
ZB = 128
G = 3
NG = BPW // G


def _embed_body(x_ref, w_ref, b_ref, o_ref):
    acc = jnp.dot(x_ref[:], w_ref[:], preferred_element_type=jnp.float32)
    o_ref[:] = jnp.maximum(acc + b_ref[:], 0.0).astype(jnp.bfloat16)


def _embed(x, w_emb, b_emb):
    return pl.pallas_call(
        _embed_body,
        out_shape=jax.ShapeDtypeStruct((N, H), jnp.bfloat16),
    )(x, w_emb, b_emb.reshape(1, H))


def _sc_body(eil_hbm, h_hbm, out_hbm, row_v, col_v, bufs, zbuf,
             agg_sh, gsem0, gsem1, ssem0, ssem1):
    cid = lax.axis_index("c")
    sid = lax.axis_index("s")
    wid = cid * NS + sid
    base = wid * BPW

    zero = jnp.zeros((32,), jnp.bfloat16)

    def zrow(i, c):
        for j in range(H // 32):
            zbuf[i, pl.ds(j * 32, 32)] = zero
        return c

    lax.fori_loop(0, ZB, zrow, 0)
    for t in range(RPS // ZB):
        pltpu.sync_copy(zbuf, agg_sh.at[pl.ds(sid * RPS + t * ZB, ZB)])

    pltpu.sync_copy(eil_hbm.at[pl.ds(base, BPW), 0], row_v.at[pl.ds(0, BPW)])
    pltpu.sync_copy(eil_hbm.at[pl.ds(base, BPW), 1], col_v.at[pl.ds(0, BPW)])

    @pl.when(wid < NBLK - NW * BPW)
    def _():
        pltpu.sync_copy(eil_hbm.at[NW * BPW + wid, 0], row_v.at[BPW])
        pltpu.sync_copy(eil_hbm.at[NW * BPW + wid, 1], col_v.at[BPW])

    plsc.subcore_barrier()

    gsems = (gsem0, gsem1)
    ssems = (ssem0, ssem1)

    def gat_start(s, g):
        for b in range(G):
            pltpu.async_copy(h_hbm.at[col_v.at[g * G + b]], bufs.at[s, b],
                             gsems[s])

    def gat_wait(s, g):
        for b in range(G):
            pltpu.make_async_copy(h_hbm.at[col_v.at[g * G + b]],
                                  bufs.at[s, b], gsems[s]).wait()

    def scat_start(s, g):
        for b in range(G):
            pltpu.async_copy(bufs.at[s, b], agg_sh.at[row_v.at[g * G + b]],
                             ssems[s], add=True)

    def scat_wait(s, g):
        for b in range(G):
            pltpu.make_async_copy(bufs.at[s, b],
                                  agg_sh.at[row_v.at[g * G + b]],
                                  ssems[s]).wait()

    gat_start(0, 0)

    def dbl(i, c):
        ga = 2 * i
        gb = 2 * i + 1
        gat_wait(0, ga)
        scat_start(0, ga)

        @pl.when(i > 0)
        def _():
            scat_wait(1, gb - 2)

        gat_start(1, gb)
        gat_wait(1, gb)
        scat_start(1, gb)
        scat_wait(0, ga)

        @pl.when(i < NG // 2 - 1)
        def _():
            gat_start(0, ga + 2)

        return c

    lax.fori_loop(0, NG // 2, dbl, 0)
    scat_wait(1, NG - 1)

    @pl.when(wid < NBLK - NW * BPW)
    def _():
        pltpu.sync_copy(h_hbm.at[col_v.at[BPW]], bufs.at[0, 0])
        pltpu.sync_copy(bufs.at[0, 0], agg_sh.at[row_v.at[BPW]], add=True)

    plsc.subcore_barrier()

    pltpu.sync_copy(agg_sh.at[pl.ds(sid * RPS, RPS)],
                    out_hbm.at[cid, pl.ds(sid * RPS, RPS)])


def _sc_scatter(eil, h):
    mesh = plsc.VectorSubcoreMesh(core_axis_name="c", subcore_axis_name="s")
    kern = functools.partial(
        pl.kernel,
        out_type=jax.ShapeDtypeStruct((NC, NP, H), jnp.bfloat16),
        mesh=mesh,
        scratch_types=[
            pltpu.VMEM((BPW + 1, K), jnp.int32),
            pltpu.VMEM((BPW + 1, K), jnp.int32),
            pltpu.VMEM((2, G, K, H), jnp.bfloat16),
            pltpu.VMEM((ZB, H), jnp.bfloat16),
            pltpu.VMEM_SHARED((NP, H), jnp.bfloat16),
            pltpu.SemaphoreType.DMA,
            pltpu.SemaphoreType.DMA,
            pltpu.SemaphoreType.DMA,
            pltpu.SemaphoreType.DMA,
        ],
        compiler_params=pltpu.CompilerParams(use_tc_tiling_on_sc=False),
    )(_sc_body)
    return kern(eil, h)


def _heads_body(p_ref, wm_ref, bm_ref, w0_ref, b0_ref, w1_ref,
                b1_ref, w2_ref, b2_ref, o0_ref, o1_ref, o2_ref):
    agg = p_ref[0].astype(jnp.float32) + p_ref[1].astype(jnp.float32)
    t = jnp.dot(agg, wm_ref[:], preferred_element_type=jnp.float32)
    t = jnp.maximum(t + bm_ref[:], 0.0)
    ridx = lax.broadcasted_iota(jnp.int32, (NP, 1), 0)
    t = jnp.where(ridx < N, t, 0.0)
    hbar = jnp.sum(t, axis=0, keepdims=True) * (1.0 / N)
    v0 = jnp.dot(hbar, w0_ref[:], preferred_element_type=jnp.float32)
    for i in range(128):
        o0_ref[i:i + 1, :] = v0[:, 64 * i:64 * (i + 1)] + b0_ref[i:i + 1, :]
    v1 = jnp.dot(hbar, w1_ref[:], preferred_element_type=jnp.float32)
    for i in range(64):
        o1_ref[i:i + 1, :] = v1[:, 32 * i:32 * (i + 1)] + b1_ref[i:i + 1, :]
    v2 = jnp.dot(hbar, w2_ref[:], preferred_element_type=jnp.float32)
    for i in range(32):
        o2_ref[i:i + 1, :] = v2[:, 16 * i:16 * (i + 1)] + b2_ref[i:i + 1, :]


def _heads(partials, w_msg, b_msg, w0, b0, w1, b1, w2, b2):
    return pl.pallas_call(
        _heads_body,
        out_shape=(
            jax.ShapeDtypeStruct((128, 64), jnp.float32),
            jax.ShapeDtypeStruct((64, 32), jnp.float32),
            jax.ShapeDtypeStruct((32, 16), jnp.float32),
        ),
    )(partials, w_msg, b_msg.reshape(1, H), w0, b0.reshape(128, 64),
      w1, b1.reshape(64, 32), w2, b2.reshape(32, 16))


def kernel(x, edge_index, W_emb, b_emb, W_msg, b_msg, W0, b0, W1, b1, W2, b2):
    h = _embed(x, W_emb, b_emb)
    eil = edge_index.reshape(2, NBLK, K).transpose(1, 0, 2)
    partials = _sc_scatter(eil, h)
    return _heads(partials, W_msg, b_msg, W0, b0, W1, b1, W2, b2)


# speedup vs baseline: 1.1149x; 1.1149x over previous
import functools

import jax
import jax.numpy as jnp
from jax import lax
from jax.experimental import pallas as pl
from jax.experimental.pallas import tpu as pltpu
from jax.experimental.pallas import tpu_sc as plsc

N = 10000
F = 128
H = 64
E = 320000
NC = 2
NS = 16
NW = NC * NS
K = 128
NBLK = E // K
BPW = NBLK // NW
NP = 10240
RPS = NP // NS
ZB = 128
G = 6
NG = BPW // G


def _embed_body(x_ref, w_ref, b_ref, o_ref):
    acc = jnp.dot(x_ref[:], w_ref[:], preferred_element_type=jnp.float32)
    o_ref[:] = jnp.maximum(acc + b_ref[:], 0.0).astype(jnp.bfloat16)


def _embed(x, w_emb, b_emb):
    return pl.pallas_call(
        _embed_body,
        out_shape=jax.ShapeDtypeStruct((N, H), jnp.bfloat16),
    )(x, w_emb, b_emb.reshape(1, H))


def _sc_body(eil_hbm, h_hbm, out_hbm, row_v, col_v, bufs, zbuf,
             agg_sh, gsem0, gsem1, ssem0, ssem1):
    cid = lax.axis_index("c")
    sid = lax.axis_index("s")
    wid = cid * NS + sid
    base = wid * BPW

    zero = jnp.zeros((32,), jnp.bfloat16)

    def zrow(i, c):
        for j in range(H // 32):
            zbuf[i, pl.ds(j * 32, 32)] = zero
        return c

    lax.fori_loop(0, ZB, zrow, 0)
    for t in range(RPS // ZB):
        pltpu.sync_copy(zbuf, agg_sh.at[pl.ds(sid * RPS + t * ZB, ZB)])

    pltpu.sync_copy(eil_hbm.at[pl.ds(base, BPW), 0], row_v.at[pl.ds(0, BPW)])
    pltpu.sync_copy(eil_hbm.at[pl.ds(base, BPW), 1], col_v.at[pl.ds(0, BPW)])

    @pl.when(wid < NBLK - NW * BPW)
    def _():
        pltpu.sync_copy(eil_hbm.at[NW * BPW + wid, 0], row_v.at[BPW])
        pltpu.sync_copy(eil_hbm.at[NW * BPW + wid, 1], col_v.at[BPW])

    plsc.subcore_barrier()

    gsems = (gsem0, gsem1)
    ssems = (ssem0, ssem1)

    def gat_start(s, g):
        for b in range(G):
            pltpu.async_copy(h_hbm.at[col_v.at[g * G + b]], bufs.at[s, b],
                             gsems[s])

    def gat_wait(s, g):
        for b in range(G):
            pltpu.make_async_copy(h_hbm.at[col_v.at[g * G + b]],
                                  bufs.at[s, b], gsems[s]).wait()

    def scat_start(s, g):
        for b in range(G):
            pltpu.async_copy(bufs.at[s, b], agg_sh.at[row_v.at[g * G + b]],
                             ssems[s], add=True)

    def scat_wait(s, g):
        for b in range(G):
            pltpu.make_async_copy(bufs.at[s, b],
                                  agg_sh.at[row_v.at[g * G + b]],
                                  ssems[s]).wait()

    gat_start(0, 0)

    def dbl(i, c):
        ga = 2 * i
        gb = 2 * i + 1
        gat_wait(0, ga)
        scat_start(0, ga)

        @pl.when(i > 0)
        def _():
            scat_wait(1, gb - 2)

        gat_start(1, gb)
        gat_wait(1, gb)
        scat_start(1, gb)
        scat_wait(0, ga)

        @pl.when(i < (NG - 2) // 2)
        def _():
            gat_start(0, ga + 2)

        return c

    lax.fori_loop(0, NG // 2, dbl, 0)
    if NG % 2:
        gat_start(0, NG - 1)
        gat_wait(0, NG - 1)
        scat_start(0, NG - 1)
        scat_wait(1, NG - 2)
        scat_wait(0, NG - 1)
    else:
        scat_wait(1, NG - 1)

    @pl.when(wid < NBLK - NW * BPW)
    def _():
        pltpu.sync_copy(h_hbm.at[col_v.at[BPW]], bufs.at[0, 0])
        pltpu.sync_copy(bufs.at[0, 0], agg_sh.at[row_v.at[BPW]], add=True)

    plsc.subcore_barrier()

    pltpu.sync_copy(agg_sh.at[pl.ds(sid * RPS, RPS)],
                    out_hbm.at[cid, pl.ds(sid * RPS, RPS)])


def _sc_scatter(eil, h):
    mesh = plsc.VectorSubcoreMesh(core_axis_name="c", subcore_axis_name="s")
    kern = functools.partial(
        pl.kernel,
        out_type=jax.ShapeDtypeStruct((NC, NP, H), jnp.bfloat16),
        mesh=mesh,
        scratch_types=[
            pltpu.VMEM((BPW + 1, K), jnp.int32),
            pltpu.VMEM((BPW + 1, K), jnp.int32),
            pltpu.VMEM((2, G, K, H), jnp.bfloat16),
            pltpu.VMEM((ZB, H), jnp.bfloat16),
            pltpu.VMEM_SHARED((NP, H), jnp.bfloat16),
            pltpu.SemaphoreType.DMA,
            pltpu.SemaphoreType.DMA,
            pltpu.SemaphoreType.DMA,
            pltpu.SemaphoreType.DMA,
        ],
        compiler_params=pltpu.CompilerParams(use_tc_tiling_on_sc=False),
    )(_sc_body)
    return kern(eil, h)


def _heads_body(p_ref, wm2_ref, bm2_ref, w0_ref, b0_ref, w1_ref,
                b1_ref, w2_ref, b2_ref, o0_ref, o1_ref, o2_ref):
    q = p_ref[:].astype(jnp.float32)
    agg = q[0:NP // 2, :] + q[NP // 2:NP, :]
    t = jnp.dot(agg, wm2_ref[:], preferred_element_type=jnp.float32)
    t = jnp.maximum(t + bm2_ref[:], 0.0)
    ridx = lax.broadcasted_iota(jnp.int32, (NP // 2, 128), 0)
    cidx = lax.broadcasted_iota(jnp.int32, (NP // 2, 128), 1)
    node = 2 * ridx + (cidx >= H).astype(jnp.int32)
    t = jnp.where(node < N, t, 0.0)
    s = jnp.sum(t, axis=0, keepdims=True)
    hbar = (s[:, 0:H] + s[:, H:128]) * (1.0 / N)
    v0 = jnp.dot(hbar, w0_ref[:], preferred_element_type=jnp.float32)
    for i in range(128):
        o0_ref[i:i + 1, :] = v0[:, 64 * i:64 * (i + 1)] + b0_ref[i:i + 1, :]
    v1 = jnp.dot(hbar, w1_ref[:], preferred_element_type=jnp.float32)
    for i in range(64):
        o1_ref[i:i + 1, :] = v1[:, 32 * i:32 * (i + 1)] + b1_ref[i:i + 1, :]
    v2 = jnp.dot(hbar, w2_ref[:], preferred_element_type=jnp.float32)
    for i in range(32):
        o2_ref[i:i + 1, :] = v2[:, 16 * i:16 * (i + 1)] + b2_ref[i:i + 1, :]


def _heads(p_packed, w_msg, b_msg, w0, b0, w1, b1, w2, b2):
    wm2 = jnp.zeros((128, 128), jnp.float32)
    wm2 = wm2.at[0:H, 0:H].set(w_msg).at[H:128, H:128].set(w_msg)
    bm2 = jnp.concatenate([b_msg, b_msg]).reshape(1, 128)
    return pl.pallas_call(
        _heads_body,
        out_shape=(
            jax.ShapeDtypeStruct((128, 64), jnp.float32),
            jax.ShapeDtypeStruct((64, 32), jnp.float32),
            jax.ShapeDtypeStruct((32, 16), jnp.float32),
        ),
    )(p_packed, wm2, bm2, w0, b0.reshape(128, 64),
      w1, b1.reshape(64, 32), w2, b2.reshape(32, 16))


def kernel(x, edge_index, W_emb, b_emb, W_msg, b_msg, W0, b0, W1, b1, W2, b2):
    h = _embed(x, W_emb, b_emb)
    eil = edge_index.reshape(2, NBLK, K).transpose(1, 0, 2)
    partials = _sc_scatter(eil, h)
    return _heads(partials.reshape(NP, 128), W_msg, b_msg,
                  W0, b0, W1, b1, W2, b2)
